# baseline (speedup 1.0000x reference)
import functools

import jax
import jax.numpy as jnp
import numpy as np
from jax import lax
from jax.experimental import pallas as pl
from jax.experimental.pallas import tpu as pltpu
from jax.experimental.pallas import tpu_sc as plsc

S, H = 2048, 1024
NH, DH = 16, 64
FF = 2048
NUM_REC = 3
NUM_EXPERTS = 2 ** NUM_REC
RB = 256
QB = 256
KB = 256
NW = 32
RPW = S // NW


def _pcall(body, **kw):
    return pl.pallas_call(body, **kw)


def _router_body(x_ref, ek_ref, pos_ref, sw_ref, dest_ref, cnt_ref,
                 tok_ref, pp_ref, xcopy_ref):
    x = x_ref[...]
    xcopy_ref[...] = x
    scores = lax.dot_general(x, ek_ref[...], (((1,), (1,)), ((), ())),
                             preferred_element_type=jnp.float32)
    m = jnp.max(scores, axis=-1, keepdims=True)
    w = 1.0 / jnp.sum(jnp.exp(scores - m), axis=-1, keepdims=True)
    chosen = jnp.argmax(scores, axis=-1).astype(jnp.int32)
    bits = lax.broadcasted_iota(jnp.int32, (S, NUM_EXPERTS), 1)
    active = ((chosen[:, None] >> bits) & 1).astype(jnp.float32)
    sw_ref[...] = active * w

    r_io = lax.broadcasted_iota(jnp.int32, (S, S), 0)
    c_io = lax.broadcasted_iota(jnp.int32, (S, S), 1)
    tri = (r_io >= c_io).astype(jnp.bfloat16)
    rank = lax.dot_general(tri, active.astype(jnp.bfloat16),
                           (((1,), (0,)), ((), ())),
                           preferred_element_type=jnp.float32)
    total = rank[S - 1:S, :]
    t_col = lax.broadcasted_iota(jnp.int32, (S, 1), 0).astype(jnp.float32)
    dest = jnp.where(active > 0.5, rank - 1.0, total + t_col - rank)
    desti = dest.astype(jnp.int32)
    dest_ref[...] = desti
    cnt_ref[...] = total.astype(jnp.int32)

    p_row = lax.broadcasted_iota(jnp.int32, (1, S), 1)
    posf = pos_ref[...].astype(jnp.float32)
    tp_in = jnp.concatenate([t_col, posf], axis=1)
    toks = []
    pps = []
    for d in range(NUM_REC):
        pm = (desti[:, d:d + 1] == p_row).astype(jnp.float32)
        tp = lax.dot_general(pm, tp_in, (((0,), (0,)), ((), ())),
                             preferred_element_type=jnp.float32)
        toks.append(tp[:, 0:1])
        pps.append(tp[:, 1:2])
    pad = jnp.zeros((S, NUM_EXPERTS - NUM_REC), jnp.float32)
    tok_ref[...] = jnp.concatenate(toks + [pad], axis=1).astype(jnp.int32)
    pp_ref[...] = jnp.concatenate(pps + [pad], axis=1).astype(jnp.int32)


def _sc_scatter(src, idx):
    mesh = plsc.VectorSubcoreMesh(core_axis_name="c", subcore_axis_name="s")

    @functools.partial(
        pl.kernel, mesh=mesh,
        out_type=jax.ShapeDtypeStruct((S, H), jnp.float32),
        scratch_types=[
            pltpu.VMEM((RPW,), jnp.int32),
            pltpu.VMEM((RPW, H), jnp.float32),
            pltpu.SemaphoreType.DMA,
        ],
    )
    def k(src_hbm, idx_hbm, out_hbm, idx_v, rows_v, sem):
        wid = lax.axis_index("s") * 2 + lax.axis_index("c")
        base = wid * RPW
        pltpu.sync_copy(idx_hbm.at[pl.ds(base, RPW)], idx_v)
        pltpu.sync_copy(src_hbm.at[pl.ds(base, RPW)], rows_v)
        pltpu.async_copy(rows_v, out_hbm.at[idx_v], sem).wait()

    return k(src, idx)


def _sc_gather(src, idx):
    mesh = plsc.VectorSubcoreMesh(core_axis_name="c", subcore_axis_name="s")

    @functools.partial(
        pl.kernel, mesh=mesh,
        out_type=jax.ShapeDtypeStruct((S, H), jnp.float32),
        scratch_types=[
            pltpu.VMEM((RPW,), jnp.int32),
            pltpu.VMEM((RPW, H), jnp.float32),
            pltpu.SemaphoreType.DMA,
        ],
    )
    def k(src_hbm, idx_hbm, out_hbm, idx_v, rows_v, sem):
        wid = lax.axis_index("s") * 2 + lax.axis_index("c")
        base = wid * RPW
        pltpu.sync_copy(idx_hbm.at[pl.ds(base, RPW)], idx_v)
        pltpu.async_copy(src_hbm.at[idx_v], rows_v, sem).wait()
        pltpu.sync_copy(rows_v, out_hbm.at[pl.ds(base, RPW)])

    return k(src, idx)


def _qkv_body(depth, cnt_ref, x_ref, ek_ref, norm_ref, pp_ref,
              wq_ref, wk_ref, wv_ref, q_ref, k_ref, v_ref):
    i = pl.program_id(0)
    a = cnt_ref[depth]

    @pl.when(i * RB < a)
    def _():
        x = x_ref[...] + ek_ref[...]
        h = x * lax.rsqrt(jnp.mean(x * x, axis=-1, keepdims=True) + 1e-6)
        h = (h * norm_ref[...]).astype(jnp.bfloat16)
        q = jnp.dot(h, wq_ref[...], preferred_element_type=jnp.float32)
        k = jnp.dot(h, wk_ref[...], preferred_element_type=jnp.float32)
        v = jnp.dot(h, wv_ref[...], preferred_element_type=jnp.float32)

        pos_blk = pp_ref[...].astype(jnp.float32)
        col = lax.broadcasted_iota(jnp.int32, (1, H), 1)
        offs = col % DH
        f = (offs % (DH // 2)).astype(jnp.float32)
        inv = jnp.exp(f * (-np.log(10000.0) / (DH // 2)))
        ang = pos_blk * inv
        cosf = jnp.cos(ang)
        sinf = jnp.sin(ang)
        first_half = offs < (DH // 2)

        def rope(t):
            rot_m = jnp.concatenate([t[:, DH // 2:], t[:, :DH // 2]], axis=1)
            rot_p = jnp.concatenate([t[:, -(DH // 2):], t[:, :-(DH // 2)]],
                                    axis=1)
            rot = jnp.where(first_half, -rot_m, rot_p)
            return t * cosf + rot * sinf

        q_ref[...] = (rope(q) * (1.0 / np.sqrt(DH))).astype(jnp.bfloat16)
        k_ref[...] = rope(k).astype(jnp.bfloat16)
        v_ref[...] = v.astype(jnp.bfloat16)

    @pl.when(i * RB >= a)
    def _():
        q_ref[...] = jnp.zeros_like(q_ref)
        k_ref[...] = jnp.zeros_like(k_ref)
        v_ref[...] = jnp.zeros_like(v_ref)


def _attn_body(depth, cnt_ref, q_ref, k_ref, v_ref, o_ref,
               acc_ref, m_ref, l_ref):
    qb = pl.program_id(1)
    kb = pl.program_id(2)
    a = cnt_ref[depth]
    last = jnp.maximum((a + KB - 1) // KB - 1, 0)

    @pl.when((qb * QB < a) & (kb * KB < a))
    def _():
        @pl.when(kb == 0)
        def _():
            acc_ref[...] = jnp.zeros_like(acc_ref)
            m_ref[...] = jnp.full_like(m_ref, -1e30)
            l_ref[...] = jnp.zeros_like(l_ref)

        key_ok = kb * KB + lax.broadcasted_iota(jnp.int32, (1, KB), 1) < a
        for sub in range(2):
            sl = slice(sub * DH, (sub + 1) * DH)
            kblk = k_ref[pl.ds(kb * KB, KB), sl]
            s = lax.dot_general(q_ref[:, sl], kblk, (((1,), (1,)), ((), ())),
                                preferred_element_type=jnp.float32)
            s = jnp.where(key_ok, s, -1e30)
            m_old = m_ref[sub]
            m_new = jnp.maximum(m_old, jnp.max(s, axis=-1, keepdims=True))
            p = jnp.exp(s - m_new)
            corr = jnp.exp(m_old - m_new)
            l_ref[sub] = l_ref[sub] * corr + jnp.sum(p, axis=-1, keepdims=True)
            pv = jnp.dot(p.astype(jnp.bfloat16), v_ref[pl.ds(kb * KB, KB), sl],
                         preferred_element_type=jnp.float32)
            acc_ref[:, sl] = acc_ref[:, sl] * corr + pv
            m_ref[sub] = m_new

    @pl.when((qb * QB < a) & (kb == last))
    def _():
        for sub in range(2):
            sl = slice(sub * DH, (sub + 1) * DH)
            o_ref[:, sl] = (acc_ref[:, sl] / l_ref[sub]).astype(jnp.bfloat16)

    @pl.when((qb * QB >= a) & (kb == 0))
    def _():
        o_ref[...] = jnp.zeros_like(o_ref)


def _post_body(depth, cnt_ref, x_ref, ek_ref, o_ref, wo_ref, norm_ref,
               wg_ref, wu_ref, wd_ref, y_ref):
    i = pl.program_id(0)
    a = cnt_ref[depth]

    @pl.when(i * RB < a)
    def _():
        x = x_ref[...] + ek_ref[...] + jnp.dot(
            o_ref[...], wo_ref[...], preferred_element_type=jnp.float32)
        h2 = x * lax.rsqrt(jnp.mean(x * x, axis=-1, keepdims=True) + 1e-6)
        h2 = (h2 * norm_ref[...]).astype(jnp.bfloat16)
        g = jnp.dot(h2, wg_ref[...], preferred_element_type=jnp.float32)
        u = jnp.dot(h2, wu_ref[...], preferred_element_type=jnp.float32)
        act = ((g * lax.logistic(g)) * u).astype(jnp.bfloat16)
        y_ref[...] = x + jnp.dot(act, wd_ref[...],
                                 preferred_element_type=jnp.float32)

    @pl.when(i * RB >= a)
    def _():
        y_ref[...] = jnp.zeros_like(y_ref)


def _combine_body(final_ref, y_ref, sw_ref, out_ref):
    out_ref[...] = final_ref[...] + y_ref[...] * sw_ref[...]


def kernel(hidden_states, position_ids, expert_keys, params):
    Bb, Ss, Hh = hidden_states.shape
    flat = hidden_states.reshape(Ss, Hh)
    pos = position_ids.reshape(Ss, 1).astype(jnp.int32)

    sw, dest3, cnt, tok3, pp3, xcopy = _pcall(
        _router_body,
        out_shape=(
            jax.ShapeDtypeStruct((S, NUM_EXPERTS), jnp.float32),
            jax.ShapeDtypeStruct((S, NUM_EXPERTS), jnp.int32),
            jax.ShapeDtypeStruct((1, NUM_EXPERTS), jnp.int32),
            jax.ShapeDtypeStruct((S, NUM_EXPERTS), jnp.int32),
            jax.ShapeDtypeStruct((S, NUM_EXPERTS), jnp.int32),
            jax.ShapeDtypeStruct((S, H), jnp.float32),
        ),
    )(flat, expert_keys, pos)
    cnt_flat = cnt.reshape(NUM_EXPERTS)

    final = flat
    for d in range(NUM_REC):
        p = params[d]
        ek_row = expert_keys[1 << d][None, :]
        wq, wk, wv, wo, wg, wu, wd = (
            p[n].astype(jnp.bfloat16)
            for n in ("Wq", "Wk", "Wv", "Wo", "Wg", "Wu", "Wd"))
        dest_d = dest3[:, d]
        pp_col = pp3[:, d:d + 1]

        xpack = _sc_scatter(final, dest_d)

        q, k, v = pl.pallas_call(
            functools.partial(_qkv_body, d),
            grid_spec=pltpu.PrefetchScalarGridSpec(
                num_scalar_prefetch=1,
                grid=(S // RB,),
                in_specs=[
                    pl.BlockSpec((RB, H), lambda i, c: (i, 0)),
                    pl.BlockSpec((1, H), lambda i, c: (0, 0)),
                    pl.BlockSpec((1, H), lambda i, c: (0, 0)),
                    pl.BlockSpec((RB, 1), lambda i, c: (i, 0)),
                    pl.BlockSpec((H, H), lambda i, c: (0, 0)),
                    pl.BlockSpec((H, H), lambda i, c: (0, 0)),
                    pl.BlockSpec((H, H), lambda i, c: (0, 0)),
                ],
                out_specs=[
                    pl.BlockSpec((RB, H), lambda i, c: (i, 0)),
                    pl.BlockSpec((RB, H), lambda i, c: (i, 0)),
                    pl.BlockSpec((RB, H), lambda i, c: (i, 0)),
                ],
            ),
            out_shape=[jax.ShapeDtypeStruct((S, H), jnp.bfloat16)] * 3,
        )(cnt_flat, xpack, ek_row, p["attn_norm"][None, :], pp_col,
          wq, wk, wv)

        o = pl.pallas_call(
            functools.partial(_attn_body, d),
            grid_spec=pltpu.PrefetchScalarGridSpec(
                num_scalar_prefetch=1,
                grid=(NH // 2, S // QB, S // KB),
                in_specs=[
                    pl.BlockSpec((QB, 2 * DH), lambda h, qb, kb, c: (qb, h)),
                    pl.BlockSpec((S, 2 * DH), lambda h, qb, kb, c: (0, h)),
                    pl.BlockSpec((S, 2 * DH), lambda h, qb, kb, c: (0, h)),
                ],
                out_specs=pl.BlockSpec((QB, 2 * DH),
                                       lambda h, qb, kb, c: (qb, h)),
                scratch_shapes=[
                    pltpu.VMEM((QB, 2 * DH), jnp.float32),
                    pltpu.VMEM((2, QB, 1), jnp.float32),
                    pltpu.VMEM((2, QB, 1), jnp.float32),
                ],
            ),
            out_shape=jax.ShapeDtypeStruct((S, H), jnp.bfloat16),
        )(cnt_flat, q, k, v)

        ypack = pl.pallas_call(
            functools.partial(_post_body, d),
            grid_spec=pltpu.PrefetchScalarGridSpec(
                num_scalar_prefetch=1,
                grid=(S // RB,),
                in_specs=[
                    pl.BlockSpec((RB, H), lambda i, c: (i, 0)),
                    pl.BlockSpec((1, H), lambda i, c: (0, 0)),
                    pl.BlockSpec((RB, H), lambda i, c: (i, 0)),
                    pl.BlockSpec((H, H), lambda i, c: (0, 0)),
                    pl.BlockSpec((1, H), lambda i, c: (0, 0)),
                    pl.BlockSpec((H, FF), lambda i, c: (0, 0)),
                    pl.BlockSpec((H, FF), lambda i, c: (0, 0)),
                    pl.BlockSpec((FF, H), lambda i, c: (0, 0)),
                ],
                out_specs=pl.BlockSpec((RB, H), lambda i, c: (i, 0)),
            ),
            out_shape=jax.ShapeDtypeStruct((S, H), jnp.float32),
        )(cnt_flat, xpack, ek_row, o, wo, p["mlp_norm"][None, :], wg, wu, wd)

        y = _sc_gather(ypack, dest_d)

        final = _pcall(
            _combine_body,
            grid=(S // 512,),
            in_specs=[
                pl.BlockSpec((512, H), lambda i: (i, 0)),
                pl.BlockSpec((512, H), lambda i: (i, 0)),
                pl.BlockSpec((512, 1), lambda i: (i, 0)),
            ],
            out_specs=pl.BlockSpec((512, H), lambda i: (i, 0)),
            out_shape=jax.ShapeDtypeStruct((S, H), jnp.float32),
        )(final, y, sw[:, d:d + 1])

    return final.reshape(Bb, Ss, Hh)


# speedup vs baseline: 1.1805x; 1.1805x over previous
import functools

import jax
import jax.numpy as jnp
import numpy as np
from jax import lax
from jax.experimental import pallas as pl
from jax.experimental.pallas import tpu as pltpu
from jax.experimental.pallas import tpu_sc as plsc

S, H = 2048, 1024
NH, DH = 16, 64
FF = 2048
NUM_REC = 3
NUM_EXPERTS = 2 ** NUM_REC
RB = 256
QB = 256
KB = 256
NW = 32
RPW = S // NW


def _pcall(body, **kw):
    return pl.pallas_call(body, **kw)


def _router_body(x_ref, ek_ref, pos_ref, sw_ref, dest_ref, cnt_ref,
                 tok_ref, pp_ref, xcopy_ref):
    x = x_ref[...]
    xcopy_ref[...] = x
    scores = lax.dot_general(x, ek_ref[...], (((1,), (1,)), ((), ())),
                             preferred_element_type=jnp.float32)
    m = jnp.max(scores, axis=-1, keepdims=True)
    w = 1.0 / jnp.sum(jnp.exp(scores - m), axis=-1, keepdims=True)
    chosen = jnp.argmax(scores, axis=-1).astype(jnp.int32)
    bits = lax.broadcasted_iota(jnp.int32, (S, NUM_EXPERTS), 1)
    active = ((chosen[:, None] >> bits) & 1).astype(jnp.float32)
    sw_ref[...] = active * w

    r_io = lax.broadcasted_iota(jnp.int32, (S, S), 0)
    c_io = lax.broadcasted_iota(jnp.int32, (S, S), 1)
    tri = (r_io >= c_io).astype(jnp.bfloat16)
    rank = lax.dot_general(tri, active.astype(jnp.bfloat16),
                           (((1,), (0,)), ((), ())),
                           preferred_element_type=jnp.float32)
    total = rank[S - 1:S, :]
    t_col = lax.broadcasted_iota(jnp.int32, (S, 1), 0).astype(jnp.float32)
    dest = jnp.where(active > 0.5, rank - 1.0, total + t_col - rank)
    desti = dest.astype(jnp.int32)
    dest_ref[...] = desti
    cnt_ref[...] = total.astype(jnp.int32)

    p_row = lax.broadcasted_iota(jnp.int32, (1, S), 1)
    posf = pos_ref[...].astype(jnp.float32)
    tp_in = jnp.concatenate([t_col, posf], axis=1)
    toks = []
    pps = []
    for d in range(NUM_REC):
        pm = (desti[:, d:d + 1] == p_row).astype(jnp.float32)
        tp = lax.dot_general(pm, tp_in, (((0,), (0,)), ((), ())),
                             preferred_element_type=jnp.float32)
        toks.append(tp[:, 0:1])
        pps.append(tp[:, 1:2])
    pad = jnp.zeros((S, NUM_EXPERTS - NUM_REC), jnp.float32)
    tok_ref[...] = jnp.concatenate(toks + [pad], axis=1).astype(jnp.int32)
    pp_ref[...] = jnp.concatenate(pps + [pad], axis=1).astype(jnp.int32)


def _sc_scatter(src, idx):
    mesh = plsc.VectorSubcoreMesh(core_axis_name="c", subcore_axis_name="s")

    @functools.partial(
        pl.kernel, mesh=mesh,
        out_type=jax.ShapeDtypeStruct((S, H), jnp.float32),
        scratch_types=[
            pltpu.VMEM((RPW,), jnp.int32),
            pltpu.VMEM((RPW, H), jnp.float32),
            pltpu.SemaphoreType.DMA,
        ],
    )
    def k(src_hbm, idx_hbm, out_hbm, idx_v, rows_v, sem):
        wid = lax.axis_index("s") * 2 + lax.axis_index("c")
        base = wid * RPW
        pltpu.sync_copy(idx_hbm.at[pl.ds(base, RPW)], idx_v)
        pltpu.sync_copy(src_hbm.at[pl.ds(base, RPW)], rows_v)
        pltpu.async_copy(rows_v, out_hbm.at[idx_v], sem).wait()

    return k(src, idx)


def _sc_gather(src, idx):
    mesh = plsc.VectorSubcoreMesh(core_axis_name="c", subcore_axis_name="s")

    @functools.partial(
        pl.kernel, mesh=mesh,
        out_type=jax.ShapeDtypeStruct((S, H), jnp.float32),
        scratch_types=[
            pltpu.VMEM((RPW,), jnp.int32),
            pltpu.VMEM((RPW, H), jnp.float32),
            pltpu.SemaphoreType.DMA,
        ],
    )
    def k(src_hbm, idx_hbm, out_hbm, idx_v, rows_v, sem):
        wid = lax.axis_index("s") * 2 + lax.axis_index("c")
        base = wid * RPW
        pltpu.sync_copy(idx_hbm.at[pl.ds(base, RPW)], idx_v)
        pltpu.async_copy(src_hbm.at[idx_v], rows_v, sem).wait()
        pltpu.sync_copy(rows_v, out_hbm.at[pl.ds(base, RPW)])

    return k(src, idx)


def _qkv_body(depth, cnt_ref, x_ref, ek_ref, norm_ref, pp_ref,
              wq_ref, wk_ref, wv_ref, q_ref, k_ref, v_ref):
    i = pl.program_id(0)
    a = cnt_ref[depth]

    @pl.when(i * RB < a)
    def _():
        x = x_ref[...] + ek_ref[...]
        h = x * lax.rsqrt(jnp.mean(x * x, axis=-1, keepdims=True) + 1e-6)
        h = (h * norm_ref[...]).astype(jnp.bfloat16)
        q = jnp.dot(h, wq_ref[...], preferred_element_type=jnp.float32)
        k = jnp.dot(h, wk_ref[...], preferred_element_type=jnp.float32)
        v = jnp.dot(h, wv_ref[...], preferred_element_type=jnp.float32)

        pos_blk = pp_ref[...].astype(jnp.float32)
        col = lax.broadcasted_iota(jnp.int32, (1, H), 1)
        offs = col % DH
        f = (offs % (DH // 2)).astype(jnp.float32)
        inv = jnp.exp(f * (-np.log(10000.0) / (DH // 2)))
        ang = pos_blk * inv
        cosf = jnp.cos(ang)
        sinf = jnp.sin(ang)
        first_half = offs < (DH // 2)

        def rope(t):
            rot_m = jnp.concatenate([t[:, DH // 2:], t[:, :DH // 2]], axis=1)
            rot_p = jnp.concatenate([t[:, -(DH // 2):], t[:, :-(DH // 2)]],
                                    axis=1)
            rot = jnp.where(first_half, -rot_m, rot_p)
            return t * cosf + rot * sinf

        q_ref[...] = (rope(q) * (1.0 / np.sqrt(DH))).astype(jnp.bfloat16)
        k_ref[...] = rope(k).astype(jnp.bfloat16)
        v_ref[...] = v.astype(jnp.bfloat16)

    @pl.when(i * RB >= a)
    def _():
        q_ref[...] = jnp.zeros_like(q_ref)
        k_ref[...] = jnp.zeros_like(k_ref)
        v_ref[...] = jnp.zeros_like(v_ref)


def _attn_body(depth, cnt_ref, q_ref, k_ref, v_ref, o_ref):
    qb = pl.program_id(1)
    a = cnt_ref[depth]
    nkb = (a + KB - 1) // KB

    @pl.when(qb * QB < a)
    def _():
        for sub in range(2):
            sl = slice(sub * DH, (sub + 1) * DH)
            q = q_ref[:, sl]

            def body(kb, carry):
                m_old, l_old, acc = carry
                kblk = k_ref[pl.ds(kb * KB, KB), sl]
                s = lax.dot_general(q, kblk, (((1,), (1,)), ((), ())),
                                    preferred_element_type=jnp.float32)
                key_ok = (kb * KB
                          + lax.broadcasted_iota(jnp.int32, (1, KB), 1)) < a
                s = jnp.where(key_ok, s, -1e30)
                m_new = jnp.maximum(m_old, jnp.max(s, axis=-1, keepdims=True))
                p = jnp.exp(s - m_new)
                corr = jnp.exp(m_old - m_new)
                l_new = l_old * corr + jnp.sum(p, axis=-1, keepdims=True)
                pv = jnp.dot(p.astype(jnp.bfloat16),
                             v_ref[pl.ds(kb * KB, KB), sl],
                             preferred_element_type=jnp.float32)
                return m_new, l_new, acc * corr + pv

            m0 = jnp.full((QB, 1), -1e30, jnp.float32)
            l0 = jnp.zeros((QB, 1), jnp.float32)
            a0 = jnp.zeros((QB, DH), jnp.float32)
            _, l, acc = lax.fori_loop(0, nkb, body, (m0, l0, a0))
            o_ref[:, sl] = (acc / l).astype(jnp.bfloat16)

    @pl.when(qb * QB >= a)
    def _():
        o_ref[...] = jnp.zeros_like(o_ref)


def _post_body(depth, cnt_ref, x_ref, ek_ref, o_ref, wo_ref, norm_ref,
               wg_ref, wu_ref, wd_ref, y_ref):
    i = pl.program_id(0)
    a = cnt_ref[depth]

    @pl.when(i * RB < a)
    def _():
        x = x_ref[...] + ek_ref[...] + jnp.dot(
            o_ref[...], wo_ref[...], preferred_element_type=jnp.float32)
        h2 = x * lax.rsqrt(jnp.mean(x * x, axis=-1, keepdims=True) + 1e-6)
        h2 = (h2 * norm_ref[...]).astype(jnp.bfloat16)
        g = jnp.dot(h2, wg_ref[...], preferred_element_type=jnp.float32)
        u = jnp.dot(h2, wu_ref[...], preferred_element_type=jnp.float32)
        act = ((g * lax.logistic(g)) * u).astype(jnp.bfloat16)
        y_ref[...] = x + jnp.dot(act, wd_ref[...],
                                 preferred_element_type=jnp.float32)

    @pl.when(i * RB >= a)
    def _():
        y_ref[...] = jnp.zeros_like(y_ref)


def _combine_body(final_ref, y_ref, sw_ref, out_ref):
    out_ref[...] = final_ref[...] + y_ref[...] * sw_ref[...]


def kernel(hidden_states, position_ids, expert_keys, params):
    Bb, Ss, Hh = hidden_states.shape
    flat = hidden_states.reshape(Ss, Hh)
    pos = position_ids.reshape(Ss, 1).astype(jnp.int32)

    sw, dest3, cnt, tok3, pp3, xcopy = _pcall(
        _router_body,
        out_shape=(
            jax.ShapeDtypeStruct((S, NUM_EXPERTS), jnp.float32),
            jax.ShapeDtypeStruct((S, NUM_EXPERTS), jnp.int32),
            jax.ShapeDtypeStruct((1, NUM_EXPERTS), jnp.int32),
            jax.ShapeDtypeStruct((S, NUM_EXPERTS), jnp.int32),
            jax.ShapeDtypeStruct((S, NUM_EXPERTS), jnp.int32),
            jax.ShapeDtypeStruct((S, H), jnp.float32),
        ),
    )(flat, expert_keys, pos)
    cnt_flat = cnt.reshape(NUM_EXPERTS)

    final = flat
    for d in range(NUM_REC):
        p = params[d]
        ek_row = expert_keys[1 << d][None, :]
        wq, wk, wv, wo, wg, wu, wd = (
            p[n].astype(jnp.bfloat16)
            for n in ("Wq", "Wk", "Wv", "Wo", "Wg", "Wu", "Wd"))
        dest_d = dest3[:, d]
        pp_col = pp3[:, d:d + 1]

        xpack = _sc_scatter(final, dest_d)

        q, k, v = pl.pallas_call(
            functools.partial(_qkv_body, d),
            grid_spec=pltpu.PrefetchScalarGridSpec(
                num_scalar_prefetch=1,
                grid=(S // RB,),
                in_specs=[
                    pl.BlockSpec((RB, H), lambda i, c: (i, 0)),
                    pl.BlockSpec((1, H), lambda i, c: (0, 0)),
                    pl.BlockSpec((1, H), lambda i, c: (0, 0)),
                    pl.BlockSpec((RB, 1), lambda i, c: (i, 0)),
                    pl.BlockSpec((H, H), lambda i, c: (0, 0)),
                    pl.BlockSpec((H, H), lambda i, c: (0, 0)),
                    pl.BlockSpec((H, H), lambda i, c: (0, 0)),
                ],
                out_specs=[
                    pl.BlockSpec((RB, H), lambda i, c: (i, 0)),
                    pl.BlockSpec((RB, H), lambda i, c: (i, 0)),
                    pl.BlockSpec((RB, H), lambda i, c: (i, 0)),
                ],
            ),
            out_shape=[jax.ShapeDtypeStruct((S, H), jnp.bfloat16)] * 3,
        )(cnt_flat, xpack, ek_row, p["attn_norm"][None, :], pp_col,
          wq, wk, wv)

        o = pl.pallas_call(
            functools.partial(_attn_body, d),
            grid_spec=pltpu.PrefetchScalarGridSpec(
                num_scalar_prefetch=1,
                grid=(NH // 2, S // QB),
                in_specs=[
                    pl.BlockSpec((QB, 2 * DH), lambda h, qb, c: (qb, h)),
                    pl.BlockSpec((S, 2 * DH), lambda h, qb, c: (0, h)),
                    pl.BlockSpec((S, 2 * DH), lambda h, qb, c: (0, h)),
                ],
                out_specs=pl.BlockSpec((QB, 2 * DH), lambda h, qb, c: (qb, h)),
            ),
            out_shape=jax.ShapeDtypeStruct((S, H), jnp.bfloat16),
        )(cnt_flat, q, k, v)

        ypack = pl.pallas_call(
            functools.partial(_post_body, d),
            grid_spec=pltpu.PrefetchScalarGridSpec(
                num_scalar_prefetch=1,
                grid=(S // RB,),
                in_specs=[
                    pl.BlockSpec((RB, H), lambda i, c: (i, 0)),
                    pl.BlockSpec((1, H), lambda i, c: (0, 0)),
                    pl.BlockSpec((RB, H), lambda i, c: (i, 0)),
                    pl.BlockSpec((H, H), lambda i, c: (0, 0)),
                    pl.BlockSpec((1, H), lambda i, c: (0, 0)),
                    pl.BlockSpec((H, FF), lambda i, c: (0, 0)),
                    pl.BlockSpec((H, FF), lambda i, c: (0, 0)),
                    pl.BlockSpec((FF, H), lambda i, c: (0, 0)),
                ],
                out_specs=pl.BlockSpec((RB, H), lambda i, c: (i, 0)),
            ),
            out_shape=jax.ShapeDtypeStruct((S, H), jnp.float32),
        )(cnt_flat, xpack, ek_row, o, wo, p["mlp_norm"][None, :], wg, wu, wd)

        y = _sc_gather(ypack, dest_d)

        final = _pcall(
            _combine_body,
            grid=(S // 512,),
            in_specs=[
                pl.BlockSpec((512, H), lambda i: (i, 0)),
                pl.BlockSpec((512, H), lambda i: (i, 0)),
                pl.BlockSpec((512, 1), lambda i: (i, 0)),
            ],
            out_specs=pl.BlockSpec((512, H), lambda i: (i, 0)),
            out_shape=jax.ShapeDtypeStruct((S, H), jnp.float32),
        )(final, y, sw[:, d:d + 1])

    return final.reshape(Bb, Ss, Hh)


# speedup vs baseline: 1.9290x; 1.6339x over previous
import functools

import jax
import jax.numpy as jnp
import numpy as np
from jax import lax
from jax.experimental import pallas as pl
from jax.experimental.pallas import tpu as pltpu
from jax.experimental.pallas import tpu_sc as plsc

S, H = 2048, 1024
NH, DH = 16, 64
FF = 2048
NUM_REC = 3
NUM_EXPERTS = 2 ** NUM_REC
RB = 256
QB = 256
KB = 256
NW = 32
RPW = S // NW


def _pcall(body, **kw):
    return pl.pallas_call(body, **kw)


def _router_body(x_ref, ek_ref, pos_ref, sw_ref, dest_ref, cnt_ref,
                 tok_ref, pp_ref, xcopy_ref):
    x = x_ref[...]
    xcopy_ref[...] = x
    scores = lax.dot_general(x, ek_ref[...], (((1,), (1,)), ((), ())),
                             preferred_element_type=jnp.float32)
    m = jnp.max(scores, axis=-1, keepdims=True)
    w = 1.0 / jnp.sum(jnp.exp(scores - m), axis=-1, keepdims=True)
    chosen = jnp.argmax(scores, axis=-1).astype(jnp.int32)
    bits = lax.broadcasted_iota(jnp.int32, (S, NUM_EXPERTS), 1)
    active = ((chosen[:, None] >> bits) & 1).astype(jnp.float32)
    sw_ref[...] = active * w

    r_io = lax.broadcasted_iota(jnp.int32, (S, S), 0)
    c_io = lax.broadcasted_iota(jnp.int32, (S, S), 1)
    tri = (r_io >= c_io).astype(jnp.bfloat16)
    rank = lax.dot_general(tri, active.astype(jnp.bfloat16),
                           (((1,), (0,)), ((), ())),
                           preferred_element_type=jnp.float32)
    total = rank[S - 1:S, :]
    t_col = lax.broadcasted_iota(jnp.int32, (S, 1), 0).astype(jnp.float32)
    dest = jnp.where(active > 0.5, rank - 1.0, total + t_col - rank)
    desti = dest.astype(jnp.int32)
    dest_ref[...] = desti
    cnt_ref[...] = total.astype(jnp.int32)

    p_row = lax.broadcasted_iota(jnp.int32, (1, S), 1)
    posf = pos_ref[...].astype(jnp.float32)
    tp_in = jnp.concatenate([t_col, posf], axis=1)
    toks = []
    pps = []
    for d in range(NUM_REC):
        pm = (desti[:, d:d + 1] == p_row).astype(jnp.float32)
        tp = lax.dot_general(pm, tp_in, (((0,), (0,)), ((), ())),
                             preferred_element_type=jnp.float32)
        toks.append(tp[:, 0:1])
        pps.append(tp[:, 1:2])
    pad = jnp.zeros((S, NUM_EXPERTS - NUM_REC), jnp.float32)
    tok_ref[...] = jnp.concatenate(toks + [pad], axis=1).astype(jnp.int32)
    pp_ref[...] = jnp.concatenate(pps + [pad], axis=1).astype(jnp.int32)


def _sc_scatter(src, idx):
    mesh = plsc.VectorSubcoreMesh(core_axis_name="c", subcore_axis_name="s")

    @functools.partial(
        pl.kernel, mesh=mesh,
        out_type=jax.ShapeDtypeStruct((S, H), jnp.float32),
        scratch_types=[
            pltpu.VMEM((RPW,), jnp.int32),
            pltpu.VMEM((RPW, H), jnp.float32),
            pltpu.SemaphoreType.DMA,
        ],
    )
    def k(src_hbm, idx_hbm, out_hbm, idx_v, rows_v, sem):
        wid = lax.axis_index("s") * 2 + lax.axis_index("c")
        base = wid * RPW
        pltpu.sync_copy(idx_hbm.at[pl.ds(base, RPW)], idx_v)
        pltpu.sync_copy(src_hbm.at[pl.ds(base, RPW)], rows_v)
        pltpu.async_copy(rows_v, out_hbm.at[idx_v], sem).wait()

    return k(src, idx)


def _sc_gather(src, idx):
    mesh = plsc.VectorSubcoreMesh(core_axis_name="c", subcore_axis_name="s")

    @functools.partial(
        pl.kernel, mesh=mesh,
        out_type=jax.ShapeDtypeStruct((S, H), jnp.float32),
        scratch_types=[
            pltpu.VMEM((RPW,), jnp.int32),
            pltpu.VMEM((RPW, H), jnp.float32),
            pltpu.SemaphoreType.DMA,
        ],
    )
    def k(src_hbm, idx_hbm, out_hbm, idx_v, rows_v, sem):
        wid = lax.axis_index("s") * 2 + lax.axis_index("c")
        base = wid * RPW
        pltpu.sync_copy(idx_hbm.at[pl.ds(base, RPW)], idx_v)
        pltpu.async_copy(src_hbm.at[idx_v], rows_v, sem).wait()
        pltpu.sync_copy(rows_v, out_hbm.at[pl.ds(base, RPW)])

    return k(src, idx)


def _qkv_body(depth, cnt_ref, x_ref, ek_ref, norm_ref, pp_ref,
              wq_ref, wk_ref, wv_ref, q_ref, k_ref, v_ref):
    i = pl.program_id(0)
    a = cnt_ref[depth]

    @pl.when(i * RB < a)
    def _():
        x = x_ref[...] + ek_ref[...]
        h = x * lax.rsqrt(jnp.mean(x * x, axis=-1, keepdims=True) + 1e-6)
        h = (h * norm_ref[...]).astype(jnp.bfloat16)
        q = jnp.dot(h, wq_ref[...], preferred_element_type=jnp.float32)
        k = jnp.dot(h, wk_ref[...], preferred_element_type=jnp.float32)
        v = jnp.dot(h, wv_ref[...], preferred_element_type=jnp.float32)

        pos_blk = pp_ref[...].astype(jnp.float32)
        col = lax.broadcasted_iota(jnp.int32, (1, H), 1)
        offs = col % DH
        f = (offs % (DH // 2)).astype(jnp.float32)
        inv = jnp.exp(f * (-np.log(10000.0) / (DH // 2)))
        ang = pos_blk * inv
        cosf = jnp.cos(ang)
        sinf = jnp.sin(ang)
        first_half = offs < (DH // 2)

        def rope(t):
            rot_m = jnp.concatenate([t[:, DH // 2:], t[:, :DH // 2]], axis=1)
            rot_p = jnp.concatenate([t[:, -(DH // 2):], t[:, :-(DH // 2)]],
                                    axis=1)
            rot = jnp.where(first_half, -rot_m, rot_p)
            return t * cosf + rot * sinf

        q_ref[...] = (rope(q) * (1.0 / np.sqrt(DH))).astype(jnp.bfloat16)
        k_ref[...] = rope(k).astype(jnp.bfloat16)
        v_ref[...] = v.astype(jnp.bfloat16)

    @pl.when(i * RB >= a)
    def _():
        q_ref[...] = jnp.zeros_like(q_ref)
        k_ref[...] = jnp.zeros_like(k_ref)
        v_ref[...] = jnp.zeros_like(v_ref)


def _attn_body(depth, cnt_ref, q_ref, k_ref, v_ref, o_ref):
    qb = pl.program_id(1)
    a = cnt_ref[depth]
    nkb = (a + KB - 1) // KB

    def go(w):
        key_ok = lax.broadcasted_iota(jnp.int32, (1, w), 1) < a
        for sub in range(2):
            sl = slice(sub * DH, (sub + 1) * DH)
            att = lax.dot_general(
                q_ref[:, sl], k_ref[:w, sl], (((1,), (1,)), ((), ())),
                preferred_element_type=jnp.float32)
            att = jnp.where(key_ok, att, -1e30)
            mx = jnp.max(att, axis=-1, keepdims=True)
            e = jnp.exp(att - mx)
            den = jnp.sum(e, axis=-1, keepdims=True)
            o = jnp.dot(e.astype(jnp.bfloat16), v_ref[:w, sl],
                        preferred_element_type=jnp.float32)
            o_ref[:, sl] = (o / den).astype(jnp.bfloat16)

    @pl.when(qb * QB < a)
    def _():
        @pl.when(a <= S // 4)
        def _():
            go(S // 4)

        @pl.when((a > S // 4) & (a <= S // 2))
        def _():
            go(S // 2)

        @pl.when((a > S // 2) & (a <= 3 * S // 4))
        def _():
            go(3 * S // 4)

        @pl.when(a > 3 * S // 4)
        def _():
            go(S)

    @pl.when(qb * QB >= a)
    def _():
        o_ref[...] = jnp.zeros_like(o_ref)


def _post_body(depth, cnt_ref, x_ref, ek_ref, o_ref, wo_ref, norm_ref,
               wg_ref, wu_ref, wd_ref, y_ref):
    i = pl.program_id(0)
    a = cnt_ref[depth]

    @pl.when(i * RB < a)
    def _():
        x = x_ref[...] + ek_ref[...] + jnp.dot(
            o_ref[...], wo_ref[...], preferred_element_type=jnp.float32)
        h2 = x * lax.rsqrt(jnp.mean(x * x, axis=-1, keepdims=True) + 1e-6)
        h2 = (h2 * norm_ref[...]).astype(jnp.bfloat16)
        g = jnp.dot(h2, wg_ref[...], preferred_element_type=jnp.float32)
        u = jnp.dot(h2, wu_ref[...], preferred_element_type=jnp.float32)
        act = ((g * lax.logistic(g)) * u).astype(jnp.bfloat16)
        y_ref[...] = x + jnp.dot(act, wd_ref[...],
                                 preferred_element_type=jnp.float32)

    @pl.when(i * RB >= a)
    def _():
        y_ref[...] = jnp.zeros_like(y_ref)


def _combine_body(final_ref, y_ref, sw_ref, out_ref):
    out_ref[...] = final_ref[...] + y_ref[...] * sw_ref[...]


def kernel(hidden_states, position_ids, expert_keys, params):
    Bb, Ss, Hh = hidden_states.shape
    flat = hidden_states.reshape(Ss, Hh)
    pos = position_ids.reshape(Ss, 1).astype(jnp.int32)

    sw, dest3, cnt, tok3, pp3, xcopy = _pcall(
        _router_body,
        out_shape=(
            jax.ShapeDtypeStruct((S, NUM_EXPERTS), jnp.float32),
            jax.ShapeDtypeStruct((S, NUM_EXPERTS), jnp.int32),
            jax.ShapeDtypeStruct((1, NUM_EXPERTS), jnp.int32),
            jax.ShapeDtypeStruct((S, NUM_EXPERTS), jnp.int32),
            jax.ShapeDtypeStruct((S, NUM_EXPERTS), jnp.int32),
            jax.ShapeDtypeStruct((S, H), jnp.float32),
        ),
    )(flat, expert_keys, pos)
    cnt_flat = cnt.reshape(NUM_EXPERTS)

    final = flat
    for d in range(NUM_REC):
        p = params[d]
        ek_row = expert_keys[1 << d][None, :]
        wq, wk, wv, wo, wg, wu, wd = (
            p[n].astype(jnp.bfloat16)
            for n in ("Wq", "Wk", "Wv", "Wo", "Wg", "Wu", "Wd"))
        dest_d = dest3[:, d]
        pp_col = pp3[:, d:d + 1]

        xpack = _sc_scatter(final, dest_d)

        q, k, v = pl.pallas_call(
            functools.partial(_qkv_body, d),
            grid_spec=pltpu.PrefetchScalarGridSpec(
                num_scalar_prefetch=1,
                grid=(S // RB,),
                in_specs=[
                    pl.BlockSpec((RB, H), lambda i, c: (i, 0)),
                    pl.BlockSpec((1, H), lambda i, c: (0, 0)),
                    pl.BlockSpec((1, H), lambda i, c: (0, 0)),
                    pl.BlockSpec((RB, 1), lambda i, c: (i, 0)),
                    pl.BlockSpec((H, H), lambda i, c: (0, 0)),
                    pl.BlockSpec((H, H), lambda i, c: (0, 0)),
                    pl.BlockSpec((H, H), lambda i, c: (0, 0)),
                ],
                out_specs=[
                    pl.BlockSpec((RB, H), lambda i, c: (i, 0)),
                    pl.BlockSpec((RB, H), lambda i, c: (i, 0)),
                    pl.BlockSpec((RB, H), lambda i, c: (i, 0)),
                ],
            ),
            out_shape=[jax.ShapeDtypeStruct((S, H), jnp.bfloat16)] * 3,
        )(cnt_flat, xpack, ek_row, p["attn_norm"][None, :], pp_col,
          wq, wk, wv)

        o = pl.pallas_call(
            functools.partial(_attn_body, d),
            grid_spec=pltpu.PrefetchScalarGridSpec(
                num_scalar_prefetch=1,
                grid=(NH // 2, S // QB),
                in_specs=[
                    pl.BlockSpec((QB, 2 * DH), lambda h, qb, c: (qb, h)),
                    pl.BlockSpec((S, 2 * DH), lambda h, qb, c: (0, h)),
                    pl.BlockSpec((S, 2 * DH), lambda h, qb, c: (0, h)),
                ],
                out_specs=pl.BlockSpec((QB, 2 * DH), lambda h, qb, c: (qb, h)),
            ),
            out_shape=jax.ShapeDtypeStruct((S, H), jnp.bfloat16),
        )(cnt_flat, q, k, v)

        ypack = pl.pallas_call(
            functools.partial(_post_body, d),
            grid_spec=pltpu.PrefetchScalarGridSpec(
                num_scalar_prefetch=1,
                grid=(S // RB,),
                in_specs=[
                    pl.BlockSpec((RB, H), lambda i, c: (i, 0)),
                    pl.BlockSpec((1, H), lambda i, c: (0, 0)),
                    pl.BlockSpec((RB, H), lambda i, c: (i, 0)),
                    pl.BlockSpec((H, H), lambda i, c: (0, 0)),
                    pl.BlockSpec((1, H), lambda i, c: (0, 0)),
                    pl.BlockSpec((H, FF), lambda i, c: (0, 0)),
                    pl.BlockSpec((H, FF), lambda i, c: (0, 0)),
                    pl.BlockSpec((FF, H), lambda i, c: (0, 0)),
                ],
                out_specs=pl.BlockSpec((RB, H), lambda i, c: (i, 0)),
            ),
            out_shape=jax.ShapeDtypeStruct((S, H), jnp.float32),
        )(cnt_flat, xpack, ek_row, o, wo, p["mlp_norm"][None, :], wg, wu, wd)

        y = _sc_gather(ypack, dest_d)

        final = _pcall(
            _combine_body,
            grid=(S // 512,),
            in_specs=[
                pl.BlockSpec((512, H), lambda i: (i, 0)),
                pl.BlockSpec((512, H), lambda i: (i, 0)),
                pl.BlockSpec((512, 1), lambda i: (i, 0)),
            ],
            out_specs=pl.BlockSpec((512, H), lambda i: (i, 0)),
            out_shape=jax.ShapeDtypeStruct((S, H), jnp.float32),
        )(final, y, sw[:, d:d + 1])

    return final.reshape(Bb, Ss, Hh)
